# baseline (speedup 1.0000x reference)
import functools

import jax
import jax.numpy as jnp
from jax import lax
from jax.experimental import pallas as pl
from jax.experimental.pallas import tpu as pltpu
from jax.experimental.pallas import tpu_sc as plsc

_THRESHOLD = 0.5

_NC = 2
_NS = 16
_NW = _NC * _NS

_ROWS = 2 * 192 * 384
_W = 384
_ROWS_PER_W = _ROWS // _NW
_CH = 64
_N_CH = _ROWS_PER_W // _CH
_NBUF = 4
_PD = 2


def _sc_copy(x_hbm, o_hbm, bufs, in_sems, out_sems):
    wid = lax.axis_index("s") * _NC + lax.axis_index("c")
    base = wid * _ROWS_PER_W

    def start_in(i):
        return pltpu.async_copy(
            x_hbm.at[pl.ds(base + i * _CH, _CH)], bufs[i % _NBUF],
            in_sems[i % _NBUF])

    def start_out(i):
        return pltpu.async_copy(
            bufs[i % _NBUF], o_hbm.at[pl.ds(base + i * _CH, _CH)],
            out_sems[i % _NBUF])

    in_copies = [None] * _NBUF
    out_copies = [None] * _NBUF
    for i in range(_PD):
        in_copies[i % _NBUF] = start_in(i)
    for i in range(_N_CH):
        b = i % _NBUF
        pf = i + _PD
        if pf < _N_CH:
            pb = pf % _NBUF
            if pf - _NBUF >= 0:
                out_copies[pb].wait()
            in_copies[pb] = start_in(pf)
        in_copies[b].wait()
        out_copies[b] = start_out(i)
    for c in out_copies:
        if c is not None:
            c.wait()


def kernel(x, heatmap0):
    del heatmap0
    b, c, h, w = x.shape
    x2 = x.reshape(_ROWS, _W)
    mesh = plsc.VectorSubcoreMesh(core_axis_name="c", subcore_axis_name="s")
    run = functools.partial(
        pl.kernel,
        out_type=jax.ShapeDtypeStruct((_ROWS, _W), x.dtype),
        mesh=mesh,
        scratch_types=[
            [pltpu.VMEM((_CH, _W), jnp.float32)] * _NBUF,
            [pltpu.SemaphoreType.DMA] * _NBUF,
            [pltpu.SemaphoreType.DMA] * _NBUF,
        ],
    )(_sc_copy)
    out = run(x2)
    return (out.reshape(b, c, h, w), jnp.float32(_THRESHOLD))


# speedup vs baseline: 1.0922x; 1.0922x over previous
import functools

import jax
import jax.numpy as jnp
from jax import lax
from jax.experimental import pallas as pl
from jax.experimental.pallas import tpu as pltpu
from jax.experimental.pallas import tpu_sc as plsc

_THRESHOLD = 0.5

_NC = 2
_NS = 16
_NW = _NC * _NS

_ROWS = 2 * 192 * 384
_W = 384
_ROWS_PER_W = _ROWS // _NW
_CH = 128
_N_CH = _ROWS_PER_W // _CH
_NBUF = 2
_PD = 1


def _sc_copy(x_hbm, o_hbm, bufs, in_sems, out_sems):
    cid = lax.axis_index("c")
    sid = lax.axis_index("s")
    wid = sid * _NC + cid
    base = wid * _ROWS_PER_W

    def start_in(i):
        return pltpu.async_copy(
            x_hbm.at[pl.ds(base + i * _CH, _CH)], bufs[i % _NBUF].at[sid],
            in_sems[i % _NBUF])

    def start_out(i):
        return pltpu.async_copy(
            bufs[i % _NBUF].at[sid], o_hbm.at[pl.ds(base + i * _CH, _CH)],
            out_sems[i % _NBUF])

    in_copies = [None] * _NBUF
    out_copies = [None] * _NBUF
    for i in range(_PD):
        in_copies[i % _NBUF] = start_in(i)
    for i in range(_N_CH):
        b = i % _NBUF
        pf = i + _PD
        if pf < _N_CH:
            pb = pf % _NBUF
            if pf - _NBUF >= 0:
                out_copies[pb].wait()
            in_copies[pb] = start_in(pf)
        in_copies[b].wait()
        out_copies[b] = start_out(i)
    for c in out_copies:
        if c is not None:
            c.wait()


def kernel(x, heatmap0):
    del heatmap0
    b, c, h, w = x.shape
    x2 = x.reshape(_ROWS, _W)
    mesh = plsc.VectorSubcoreMesh(core_axis_name="c", subcore_axis_name="s")
    run = functools.partial(
        pl.kernel,
        out_type=jax.ShapeDtypeStruct((_ROWS, _W), x.dtype),
        mesh=mesh,
        scratch_types=[
            [pltpu.VMEM_SHARED((_NS, _CH, _W), jnp.float32)] * _NBUF,
            [pltpu.SemaphoreType.DMA] * _NBUF,
            [pltpu.SemaphoreType.DMA] * _NBUF,
        ],
    )(_sc_copy)
    out = run(x2)
    return (out.reshape(b, c, h, w), jnp.float32(_THRESHOLD))


# speedup vs baseline: 1.0967x; 1.0041x over previous
import functools

import jax
import jax.numpy as jnp
from jax import lax
from jax.experimental import pallas as pl
from jax.experimental.pallas import tpu as pltpu
from jax.experimental.pallas import tpu_sc as plsc

_THRESHOLD = 0.5

_NC = 2
_NS = 16
_NW = _NC * _NS

_ROWS = 2 * 192 * 384
_W = 384
_ROWS_PER_W = _ROWS // _NW
_CH = 144
_N_CH = _ROWS_PER_W // _CH
_NBUF = 2
_PD = 1


def _sc_copy(x_hbm, o_hbm, bufs, in_sems, out_sems):
    cid = lax.axis_index("c")
    sid = lax.axis_index("s")
    wid = sid * _NC + cid
    base = wid * _ROWS_PER_W

    def start_in(i):
        return pltpu.async_copy(
            x_hbm.at[pl.ds(base + i * _CH, _CH)], bufs[i % _NBUF].at[sid],
            in_sems[i % _NBUF])

    def start_out(i):
        return pltpu.async_copy(
            bufs[i % _NBUF].at[sid], o_hbm.at[pl.ds(base + i * _CH, _CH)],
            out_sems[i % _NBUF])

    in_copies = [None] * _NBUF
    out_copies = [None] * _NBUF
    for i in range(_PD):
        in_copies[i % _NBUF] = start_in(i)
    for i in range(_N_CH):
        b = i % _NBUF
        pf = i + _PD
        if pf < _N_CH:
            pb = pf % _NBUF
            if pf - _NBUF >= 0:
                out_copies[pb].wait()
            in_copies[pb] = start_in(pf)
        in_copies[b].wait()
        out_copies[b] = start_out(i)
    for c in out_copies:
        if c is not None:
            c.wait()


def kernel(x, heatmap0):
    del heatmap0
    b, c, h, w = x.shape
    x2 = x.reshape(_ROWS, _W)
    mesh = plsc.VectorSubcoreMesh(core_axis_name="c", subcore_axis_name="s")
    run = functools.partial(
        pl.kernel,
        out_type=jax.ShapeDtypeStruct((_ROWS, _W), x.dtype),
        mesh=mesh,
        scratch_types=[
            [pltpu.VMEM_SHARED((_NS, _CH, _W), jnp.float32)] * _NBUF,
            [pltpu.SemaphoreType.DMA] * _NBUF,
            [pltpu.SemaphoreType.DMA] * _NBUF,
        ],
    )(_sc_copy)
    out = run(x2)
    return (out.reshape(b, c, h, w), jnp.float32(_THRESHOLD))
